# baseline (speedup 1.0000x reference)
import functools

import jax
import jax.numpy as jnp
from jax import lax
from jax.experimental import pallas as pl
from jax.experimental.pallas import tpu as pltpu
from jax.experimental.pallas import tpu_sc as plsc

BATCH = 16384
EMB_DIM = 32
NUM_CORES = 2
NUM_SUBCORES = 16
NUM_WORKERS = NUM_CORES * NUM_SUBCORES
B_PER_W = BATCH // NUM_WORKERS
CHUNK = 128
N_CHUNKS = B_PER_W // CHUNK
GROUPS = B_PER_W // 16


def _sc_body(uids_ref, gids_ref, user_ref, item_ref, out_ref,
             uid_v, gid_v, ut_v, gt_v, out_v, usem, gsem):
    wid = lax.axis_index("s") * NUM_CORES + lax.axis_index("c")
    base = wid * B_PER_W

    pltpu.sync_copy(uids_ref.at[pl.ds(wid * N_CHUNKS, N_CHUNKS)], uid_v)
    pltpu.sync_copy(gids_ref.at[pl.ds(wid * N_CHUNKS, N_CHUNKS)], gid_v)

    def fire(d, carry):
        for c in range(N_CHUNKS):
            pltpu.async_copy(
                user_ref.at[d].at[uid_v.at[c]],
                ut_v.at[d, pl.ds(c * CHUNK, CHUNK)], usem)
            pltpu.async_copy(
                item_ref.at[d].at[gid_v.at[c]],
                gt_v.at[d, pl.ds(c * CHUNK, CHUNK)], gsem)
        return carry

    lax.fori_loop(0, EMB_DIM, fire, 0)

    pltpu.make_async_copy(
        user_ref.at[pl.ds(0, EMB_DIM), pl.ds(0, B_PER_W)], ut_v, usem).wait()
    pltpu.make_async_copy(
        item_ref.at[pl.ds(0, EMB_DIM), pl.ds(0, B_PER_W)], gt_v, gsem).wait()

    def group(j, carry):
        col = j * 16
        acc = jnp.zeros((16,), jnp.float32)
        for d in range(EMB_DIM):
            acc = acc + ut_v[d, pl.ds(col, 16)] * gt_v[d, pl.ds(col, 16)]
        out_v[pl.ds(col, 16)] = acc
        return carry

    lax.fori_loop(0, GROUPS, group, 0)

    pltpu.sync_copy(out_v, out_ref.at[pl.ds(base, B_PER_W)])


@jax.jit
def kernel(uids, gids, user_emb, item_emb):
    uids2d = uids.astype(jnp.int32).reshape(BATCH // CHUNK, CHUNK)
    gids2d = gids.astype(jnp.int32).reshape(BATCH // CHUNK, CHUNK)
    user_t = user_emb.T
    item_t = item_emb.T
    mesh = plsc.VectorSubcoreMesh(core_axis_name="c", subcore_axis_name="s",
                                  num_cores=NUM_CORES, num_subcores=NUM_SUBCORES)
    run = functools.partial(
        pl.kernel,
        out_type=jax.ShapeDtypeStruct((BATCH,), jnp.float32),
        mesh=mesh,
        compiler_params=pltpu.CompilerParams(use_tc_tiling_on_sc=False),
        scratch_types=[
            pltpu.VMEM((N_CHUNKS, CHUNK), jnp.int32),
            pltpu.VMEM((N_CHUNKS, CHUNK), jnp.int32),
            pltpu.VMEM((EMB_DIM, B_PER_W), jnp.float32),
            pltpu.VMEM((EMB_DIM, B_PER_W), jnp.float32),
            pltpu.VMEM((B_PER_W,), jnp.float32),
            pltpu.SemaphoreType.DMA,
            pltpu.SemaphoreType.DMA,
        ],
    )(_sc_body)
    return run(uids2d, gids2d, user_t, item_t)


# speedup vs baseline: 22.3525x; 22.3525x over previous
import functools

import jax
import jax.numpy as jnp
from jax import lax
from jax.experimental import pallas as pl
from jax.experimental.pallas import tpu as pltpu
from jax.experimental.pallas import tpu_sc as plsc

BATCH = 16384
EMB_DIM = 32
NUM_CORES = 2
NUM_SUBCORES = 16
NUM_WORKERS = NUM_CORES * NUM_SUBCORES
B_PER_W = BATCH // NUM_WORKERS
CPOS = 4
N_CHUNKS = B_PER_W // CPOS


def _iota16():
    return lax.iota(jnp.int32, 16)


def _extract(vec, j):
    return lax.reduce_sum(jnp.where(_iota16() == j, vec, 0), axes=(0,))


def _fire(tab_ref, ids_v, buf, sem, chunk):
    l0 = chunk * CPOS
    o16 = pl.multiple_of((l0 // 16) * 16, 16)
    vec = ids_v[pl.ds(o16, 16)]
    jbase = l0 - o16
    for i in range(CPOS):
        tid = _extract(vec, jbase + i)
        col = pl.multiple_of((tid >> 7) * 128, 128)
        pltpu.async_copy(
            tab_ref.at[pl.ds(0, EMB_DIM), pl.ds(col, 128)], buf.at[i], sem)


def _drain(tab_ref, buf, sem):
    for i in range(CPOS):
        pltpu.make_async_copy(
            tab_ref.at[pl.ds(0, EMB_DIM), pl.ds(0, 128)], buf.at[i], sem
        ).wait()


def _process(uids_v, gids_v, ubuf, gbuf, out_v, chunk):
    iota = _iota16()
    l0 = chunk * CPOS
    o16 = pl.multiple_of((l0 // 16) * 16, 16)
    jbase = l0 - o16
    uvec = uids_v[pl.ds(o16, 16)]
    gvec = gids_v[pl.ds(o16, 16)]
    ovec = out_v[pl.ds(o16, 16)]
    for i in range(CPOS):
        j = jbase + i
        uid = _extract(uvec, j)
        gid = _extract(gvec, j)
        ii = jnp.full((16,), i, jnp.int32)
        ulane = jnp.full((16,), uid & 127, jnp.int32)
        glane = jnp.full((16,), gid & 127, jnp.int32)
        ul = plsc.load_gather(ubuf, [ii, iota, ulane])
        uh = plsc.load_gather(ubuf, [ii, iota + 16, ulane])
        gl = plsc.load_gather(gbuf, [ii, iota, glane])
        gh = plsc.load_gather(gbuf, [ii, iota + 16, glane])
        s = lax.reduce_sum(ul * gl + uh * gh, axes=(0,))
        ovec = jnp.where(iota == j, s, ovec)
    out_v[pl.ds(o16, 16)] = ovec


def _sc_body(uids_ref, gids_ref, user_ref, item_ref, out_ref,
             uids_v, gids_v, ubuf0, ubuf1, gbuf0, gbuf1, out_v,
             usem0, usem1, gsem0, gsem1):
    wid = lax.axis_index("s") * NUM_CORES + lax.axis_index("c")
    base = wid * B_PER_W

    pltpu.sync_copy(uids_ref.at[pl.ds(base, B_PER_W)], uids_v)
    pltpu.sync_copy(gids_ref.at[pl.ds(base, B_PER_W)], gids_v)

    _fire(user_ref, uids_v, ubuf0, usem0, 0)
    _fire(item_ref, gids_v, gbuf0, gsem0, 0)
    _fire(user_ref, uids_v, ubuf1, usem1, 1)
    _fire(item_ref, gids_v, gbuf1, gsem1, 1)

    def step(k, carry):
        _drain(user_ref, ubuf0, usem0)
        _drain(item_ref, gbuf0, gsem0)
        _process(uids_v, gids_v, ubuf0, gbuf0, out_v, 2 * k)
        _fire(user_ref, uids_v, ubuf0, usem0, 2 * k + 2)
        _fire(item_ref, gids_v, gbuf0, gsem0, 2 * k + 2)
        _drain(user_ref, ubuf1, usem1)
        _drain(item_ref, gbuf1, gsem1)
        _process(uids_v, gids_v, ubuf1, gbuf1, out_v, 2 * k + 1)
        _fire(user_ref, uids_v, ubuf1, usem1, 2 * k + 3)
        _fire(item_ref, gids_v, gbuf1, gsem1, 2 * k + 3)
        return carry

    lax.fori_loop(0, N_CHUNKS // 2 - 1, step, 0)

    _drain(user_ref, ubuf0, usem0)
    _drain(item_ref, gbuf0, gsem0)
    _process(uids_v, gids_v, ubuf0, gbuf0, out_v, N_CHUNKS - 2)
    _drain(user_ref, ubuf1, usem1)
    _drain(item_ref, gbuf1, gsem1)
    _process(uids_v, gids_v, ubuf1, gbuf1, out_v, N_CHUNKS - 1)

    pltpu.sync_copy(out_v, out_ref.at[pl.ds(base, B_PER_W)])


@jax.jit
def kernel(uids, gids, user_emb, item_emb):
    uids1d = uids.astype(jnp.int32)
    gids1d = gids.astype(jnp.int32)
    user_t = user_emb.T
    item_t = item_emb.T
    mesh = plsc.VectorSubcoreMesh(core_axis_name="c", subcore_axis_name="s",
                                  num_cores=NUM_CORES, num_subcores=NUM_SUBCORES)
    run = functools.partial(
        pl.kernel,
        out_type=jax.ShapeDtypeStruct((BATCH,), jnp.float32),
        mesh=mesh,
        compiler_params=pltpu.CompilerParams(needs_layout_passes=False),
        scratch_types=[
            pltpu.VMEM((B_PER_W,), jnp.int32),
            pltpu.VMEM((B_PER_W,), jnp.int32),
            pltpu.VMEM((CPOS, EMB_DIM, 128), jnp.float32),
            pltpu.VMEM((CPOS, EMB_DIM, 128), jnp.float32),
            pltpu.VMEM((CPOS, EMB_DIM, 128), jnp.float32),
            pltpu.VMEM((CPOS, EMB_DIM, 128), jnp.float32),
            pltpu.VMEM((B_PER_W,), jnp.float32),
            pltpu.SemaphoreType.DMA,
            pltpu.SemaphoreType.DMA,
            pltpu.SemaphoreType.DMA,
            pltpu.SemaphoreType.DMA,
        ],
    )(_sc_body)
    return run(uids1d, gids1d, user_t, item_t)


# speedup vs baseline: 22.4258x; 1.0033x over previous
import functools

import jax
import jax.numpy as jnp
from jax import lax
from jax.experimental import pallas as pl
from jax.experimental.pallas import tpu as pltpu
from jax.experimental.pallas import tpu_sc as plsc

BATCH = 16384
EMB_DIM = 32
NUM_CORES = 2
NUM_SUBCORES = 16
NUM_WORKERS = NUM_CORES * NUM_SUBCORES
B_PER_W = BATCH // NUM_WORKERS
CPOS = 4
N_CHUNKS = B_PER_W // CPOS


def _iota16():
    return lax.iota(jnp.int32, 16)


def _extract(vec, j):
    return lax.reduce_sum(jnp.where(_iota16() == j, vec, 0), axes=(0,))


def _fire(tab_ref, ids_v, buf, sem, chunk):
    l0 = chunk * CPOS
    o16 = pl.multiple_of((l0 // 16) * 16, 16)
    vec = ids_v[pl.ds(o16, 16)]
    jbase = l0 - o16
    for i in range(CPOS):
        tid = _extract(vec, jbase + i)
        col = pl.multiple_of((tid >> 7) * 128, 128)
        pltpu.async_copy(
            tab_ref.at[pl.ds(0, EMB_DIM), pl.ds(col, 128)], buf.at[i], sem)


def _drain(tab_ref, buf, sem):
    for i in range(CPOS):
        pltpu.make_async_copy(
            tab_ref.at[pl.ds(0, EMB_DIM), pl.ds(0, 128)], buf.at[i], sem
        ).wait()


def _process(uids_v, gids_v, ubuf, gbuf, out_v, chunk):
    iota = _iota16()
    l0 = chunk * CPOS
    o16 = pl.multiple_of((l0 // 16) * 16, 16)
    jbase = l0 - o16
    uvec = uids_v[pl.ds(o16, 16)]
    gvec = gids_v[pl.ds(o16, 16)]
    ovec = out_v[pl.ds(o16, 16)]
    for i in range(CPOS):
        j = jbase + i
        uid = _extract(uvec, j)
        gid = _extract(gvec, j)
        ii = jnp.full((16,), i, jnp.int32)
        ulane = jnp.full((16,), uid & 127, jnp.int32)
        glane = jnp.full((16,), gid & 127, jnp.int32)
        ul = plsc.load_gather(ubuf, [ii, iota, ulane])
        uh = plsc.load_gather(ubuf, [ii, iota + 16, ulane])
        gl = plsc.load_gather(gbuf, [ii, iota, glane])
        gh = plsc.load_gather(gbuf, [ii, iota + 16, glane])
        s = lax.reduce_sum(ul * gl + uh * gh, axes=(0,))
        ovec = jnp.where(iota == j, s, ovec)
    out_v[pl.ds(o16, 16)] = ovec


def _sc_body(uids_ref, gids_ref, user_ref, item_ref, out_ref,
             uids_v, gids_v, ubuf0, ubuf1, ubuf2, gbuf0, gbuf1, gbuf2, out_v,
             usem0, usem1, usem2, gsem0, gsem1, gsem2):
    wid = lax.axis_index("s") * NUM_CORES + lax.axis_index("c")
    base = wid * B_PER_W

    ubufs = (ubuf0, ubuf1, ubuf2)
    gbufs = (gbuf0, gbuf1, gbuf2)
    usems = (usem0, usem1, usem2)
    gsems = (gsem0, gsem1, gsem2)

    pltpu.sync_copy(uids_ref.at[pl.ds(base, B_PER_W)], uids_v)
    pltpu.sync_copy(gids_ref.at[pl.ds(base, B_PER_W)], gids_v)

    _fire(user_ref, uids_v, ubuf0, usem0, 0)
    _fire(item_ref, gids_v, gbuf0, gsem0, 0)
    _fire(user_ref, uids_v, ubuf1, usem1, 1)
    _fire(item_ref, gids_v, gbuf1, gsem1, 1)

    def step(k, carry):
        for i in range(3):
            c = 3 * k + i
            _drain(user_ref, ubufs[i], usems[i])
            _drain(item_ref, gbufs[i], gsems[i])
            _process(uids_v, gids_v, ubufs[i], gbufs[i], out_v, c)
            nb = (i + 2) % 3
            _fire(user_ref, uids_v, ubufs[nb], usems[nb], c + 2)
            _fire(item_ref, gids_v, gbufs[nb], gsems[nb], c + 2)
        return carry

    lax.fori_loop(0, N_CHUNKS // 3, step, 0)

    _drain(user_ref, ubuf0, usem0)
    _drain(item_ref, gbuf0, gsem0)
    _process(uids_v, gids_v, ubuf0, gbuf0, out_v, N_CHUNKS - 2)
    _drain(user_ref, ubuf1, usem1)
    _drain(item_ref, gbuf1, gsem1)
    _process(uids_v, gids_v, ubuf1, gbuf1, out_v, N_CHUNKS - 1)

    pltpu.sync_copy(out_v, out_ref.at[pl.ds(base, B_PER_W)])


@jax.jit
def kernel(uids, gids, user_emb, item_emb):
    uids1d = uids.astype(jnp.int32)
    gids1d = gids.astype(jnp.int32)
    user_t = user_emb.T
    item_t = item_emb.T
    mesh = plsc.VectorSubcoreMesh(core_axis_name="c", subcore_axis_name="s",
                                  num_cores=NUM_CORES, num_subcores=NUM_SUBCORES)
    run = functools.partial(
        pl.kernel,
        out_type=jax.ShapeDtypeStruct((BATCH,), jnp.float32),
        mesh=mesh,
        compiler_params=pltpu.CompilerParams(needs_layout_passes=False),
        scratch_types=[
            pltpu.VMEM((B_PER_W,), jnp.int32),
            pltpu.VMEM((B_PER_W,), jnp.int32),
            pltpu.VMEM((CPOS, EMB_DIM, 128), jnp.float32),
            pltpu.VMEM((CPOS, EMB_DIM, 128), jnp.float32),
            pltpu.VMEM((CPOS, EMB_DIM, 128), jnp.float32),
            pltpu.VMEM((CPOS, EMB_DIM, 128), jnp.float32),
            pltpu.VMEM((CPOS, EMB_DIM, 128), jnp.float32),
            pltpu.VMEM((CPOS, EMB_DIM, 128), jnp.float32),
            pltpu.VMEM((B_PER_W,), jnp.float32),
            pltpu.SemaphoreType.DMA,
            pltpu.SemaphoreType.DMA,
            pltpu.SemaphoreType.DMA,
            pltpu.SemaphoreType.DMA,
            pltpu.SemaphoreType.DMA,
            pltpu.SemaphoreType.DMA,
        ],
    )(_sc_body)
    return run(uids1d, gids1d, user_t, item_t)
